# Initial kernel scaffold; baseline (speedup 1.0000x reference)
import jax
import jax.numpy as jnp
from jax.experimental import pallas as pl


def kernel(t, y, W, edge_index, perm):
    raise NotImplementedError("write your pallas kernel here")


# speedup vs baseline: 31.5782x; 31.5782x over previous
import functools

import jax
import jax.numpy as jnp
from jax import lax
from jax.experimental import pallas as pl
from jax.experimental.pallas import tpu as pltpu
from jax.experimental.pallas import tpu_sc as plsc

_N = 50000
_EH = 800000
_D = 16
_NC = 2
_NS = 16
_NW = _NC * _NS
_CHUNK = 128
_CPT = -(-_EH // (_NW * _CHUNK))
_EPT = _CPT * _CHUNK
_EP = _EPT * _NW
_RSEG = 3136
_NP = _RSEG * _NS
_SB = _RSEG // 4
_F32 = jnp.float32
_I32 = jnp.int32


def _mesh():
    return plsc.VectorSubcoreMesh(
        core_axis_name="c", subcore_axis_name="s",
        num_cores=_NC, num_subcores=_NS)


def _sc_params():
    return pltpu.CompilerParams(
        needs_layout_passes=False, use_tc_tiling_on_sc=False)


def _rsqrt16(x):
    i = lax.bitcast_convert_type(x, _I32)
    i = jnp.int32(0x5F3759DF) - (i >> 1)
    r = lax.bitcast_convert_type(i, _F32)
    for _ in range(3):
        r = r * (1.5 - 0.5 * x * r * r)
    return r


def _deg_kernel(w1, w2, srcp, dstp):
    @functools.partial(
        pl.kernel,
        out_type=(jax.ShapeDtypeStruct((_EP,), _F32),
                  jax.ShapeDtypeStruct((_NC * _NP,), _F32)),
        mesh=_mesh(),
        scratch_types=[
            pltpu.VMEM((_CHUNK,), _F32),
            pltpu.VMEM((_CHUNK,), _F32),
            pltpu.VMEM((_CHUNK,), _F32),
            pltpu.VMEM((_CHUNK,), _I32),
            pltpu.VMEM((_CHUNK,), _I32),
            pltpu.VMEM((_NP,), _F32),
            pltpu.VMEM((_RSEG,), _F32),
            pltpu.VMEM((_RSEG,), _F32),
            pltpu.VMEM_SHARED((_NS * _NP,), _F32),
        ],
        compiler_params=_sc_params(),
    )
    def k(w1_ref, w2_ref, src_ref, dst_ref, wout_ref, degp_ref,
          w1b, w2b, wb, sb, db, degtile, tmpb, accb, degsh):
        c = lax.axis_index("c")
        s = lax.axis_index("s")
        wid = c * _NS + s
        ebase = wid * _EPT
        zero16 = jnp.zeros((16,), _F32)
        iota16 = lax.iota(_I32, 16)

        @pl.loop(0, _NP, step=16)
        def _(i):
            degtile[pl.ds(i, 16)] = zero16

        @pl.loop(0, _CPT)
        def _(kk):
            base = pl.multiple_of(ebase + kk * _CHUNK, _CHUNK)
            pltpu.sync_copy(w1_ref.at[pl.ds(base, _CHUNK)], w1b)
            pltpu.sync_copy(w2_ref.at[pl.ds(base, _CHUNK)], w2b)
            pltpu.sync_copy(src_ref.at[pl.ds(base, _CHUNK)], sb)
            pltpu.sync_copy(dst_ref.at[pl.ds(base, _CHUNK)], db)

            @pl.loop(0, _CHUNK, step=16)
            def _(g):
                a = w1b[pl.ds(g, 16)]
                b = w2b[pl.ds(g, 16)]
                wv = 1.0 / (1.0 + jnp.exp(-0.5 * (a + b)))
                gidx = base + g + iota16
                wv = jnp.where(gidx < _EH, wv, 0.0)
                wb[pl.ds(g, 16)] = wv
                plsc.addupdate_scatter(degtile, [db[pl.ds(g, 16)]], wv)
                plsc.addupdate_scatter(degtile, [sb[pl.ds(g, 16)]], wv)

            pltpu.sync_copy(wb, wout_ref.at[pl.ds(base, _CHUNK)])

        pltpu.sync_copy(degtile, degsh.at[pl.ds(pl.multiple_of(s * _NP, 8), _NP)])
        plsc.subcore_barrier()
        rbase = s * _RSEG

        @pl.loop(0, _RSEG, step=16)
        def _(i):
            accb[pl.ds(i, 16)] = zero16

        @pl.loop(0, _NS)
        def _(j):
            off = pl.multiple_of(j * _NP + rbase, 8)
            pltpu.sync_copy(degsh.at[pl.ds(off, _RSEG)], tmpb)

            @pl.loop(0, _RSEG, step=16)
            def _(i):
                accb[pl.ds(i, 16)] = accb[pl.ds(i, 16)] + tmpb[pl.ds(i, 16)]

        pltpu.sync_copy(
            accb, degp_ref.at[pl.ds(pl.multiple_of(c * _NP + rbase, 8), _RSEG)])

    return k(w1, w2, srcp, dstp)


def _edge_kernel(ytp, degp, srcp, dstp, w):
    @functools.partial(
        pl.kernel,
        out_type=(jax.ShapeDtypeStruct((_NC * _NP, _D), _F32),
                  jax.ShapeDtypeStruct((_NP,), _F32)),
        mesh=_mesh(),
        scratch_types=[
            pltpu.VMEM((_RSEG,), _F32),
            pltpu.VMEM((_RSEG,), _F32),
            pltpu.VMEM((_RSEG,), _F32),
            pltpu.VMEM((_RSEG,), _F32),
            pltpu.VMEM((_SB, _D), _F32),
            pltpu.VMEM((_CHUNK,), _I32),
            pltpu.VMEM((_CHUNK,), _I32),
            pltpu.VMEM((_CHUNK,), _F32),
            pltpu.VMEM((_CHUNK, _D), _F32),
            pltpu.VMEM((_CHUNK, _D), _F32),
            pltpu.VMEM_SHARED((_NP, _D), _F32),
            pltpu.VMEM_SHARED((_NP, _D), _F32),
        ],
        compiler_params=_sc_params(),
    )
    def k(yt_ref, degp_ref, src_ref, dst_ref, w_ref, accp_ref, degc_ref,
          p0b, p1b, degb, rsb, yb, sb, db, wbuf, rows_s, rows_d, zsh, accsh):
        c = lax.axis_index("c")
        s = lax.axis_index("s")
        rbase = s * _RSEG
        zero16 = jnp.zeros((16,), _F32)

        rb8 = pl.multiple_of(rbase, 8)
        pltpu.sync_copy(degp_ref.at[pl.ds(rb8, _RSEG)], p0b)
        pltpu.sync_copy(
            degp_ref.at[pl.ds(pl.multiple_of(_NP + rbase, 8), _RSEG)], p1b)

        @pl.loop(0, _RSEG, step=16)
        def _(i):
            dv = 1.0 + p0b[pl.ds(i, 16)] + p1b[pl.ds(i, 16)]
            degb[pl.ds(i, 16)] = dv
            rsb[pl.ds(i, 16)] = _rsqrt16(dv)

        @pl.when(c == 0)
        def _():
            pltpu.sync_copy(degb, degc_ref.at[pl.ds(rb8, _RSEG)])

        @pl.loop(0, _SB)
        def _(r):
            yb[r] = zero16

        for blk in range(_RSEG // _SB):
            bb = pl.multiple_of(rbase + blk * _SB, 8)
            pltpu.sync_copy(yb, accsh.at[pl.ds(bb, _SB)])

        for blk in range(_RSEG // _SB):
            bb = pl.multiple_of(rbase + blk * _SB, 8)
            pltpu.sync_copy(yt_ref.at[pl.ds(bb, _SB)], yb)

            @pl.loop(0, _SB, step=16)
            def _(g):
                rs16 = rsb[pl.ds(blk * _SB + g, 16)]
                for j in range(16):
                    yb[g + j] = yb[g + j] * rs16[j]

            pltpu.sync_copy(yb, zsh.at[pl.ds(bb, _SB)])

        plsc.subcore_barrier()

        wid = c * _NS + s
        ebase = wid * _EPT

        @pl.loop(0, _CPT)
        def _(kk):
            base = pl.multiple_of(ebase + kk * _CHUNK, _CHUNK)
            pltpu.sync_copy(src_ref.at[pl.ds(base, _CHUNK)], sb)
            pltpu.sync_copy(dst_ref.at[pl.ds(base, _CHUNK)], db)
            pltpu.sync_copy(w_ref.at[pl.ds(base, _CHUNK)], wbuf)
            pltpu.sync_copy(zsh.at[sb], rows_s)
            pltpu.sync_copy(zsh.at[db], rows_d)

            @pl.loop(0, _CHUNK, step=16)
            def _(g):
                wv16 = wbuf[pl.ds(g, 16)]
                for j in range(16):
                    wv = wv16[j]
                    rows_s[g + j] = rows_s[g + j] * wv
                    rows_d[g + j] = rows_d[g + j] * wv

            pltpu.sync_copy(rows_s, accsh.at[db], add=True)
            pltpu.sync_copy(rows_d, accsh.at[sb], add=True)

        plsc.subcore_barrier()

        for blk in range(_RSEG // _SB):
            bb = pl.multiple_of(rbase + blk * _SB, 8)
            pltpu.sync_copy(accsh.at[pl.ds(bb, _SB)], yb)
            obb = pl.multiple_of(c * _NP + rbase + blk * _SB, 8)
            pltpu.sync_copy(yb, accp_ref.at[pl.ds(obb, _SB)])

    return k(ytp, degp, srcp, dstp, w)


def _combine(accp, degc, ytp):
    br = _NP // 8

    def body(acc_ref, deg_ref, y_ref, o_ref):
        a = acc_ref[0] + acc_ref[1]
        deg = deg_ref[...]
        rs = lax.rsqrt(deg)
        o_ref[...] = a * rs - (1.0 - 1.0 / deg) * y_ref[...]

    return pl.pallas_call(
        body,
        grid=(8,),
        in_specs=[
            pl.BlockSpec((_NC, br, _D), lambda i: (0, i, 0)),
            pl.BlockSpec((br, 1), lambda i: (i, 0)),
            pl.BlockSpec((br, _D), lambda i: (i, 0)),
        ],
        out_specs=pl.BlockSpec((br, _D), lambda i: (i, 0)),
        out_shape=jax.ShapeDtypeStruct((_NP, _D), _F32),
    )(accp.reshape(_NC, _NP, _D), degc.reshape(_NP, 1), ytp)


def kernel(t, y, W, edge_index, perm):
    del t, perm
    wf = W.reshape(-1)
    pad_e = _EP - _EH
    w1 = jnp.pad(wf[:_EH], (0, pad_e))
    w2 = jnp.pad(wf[_EH:], (0, pad_e))
    srcp = jnp.pad(edge_index[0, :_EH], (0, pad_e))
    dstp = jnp.pad(edge_index[1, :_EH], (0, pad_e))
    ytp = jnp.pad(y.T, ((0, _NP - _N), (0, 0)))
    w, degp = _deg_kernel(w1, w2, srcp, dstp)
    accp, degc = _edge_kernel(ytp, degp, srcp, dstp, w)
    res = _combine(accp, degc, ytp)
    return res[:_N].T


# speedup vs baseline: 48.2207x; 1.5270x over previous
import functools

import jax
import jax.numpy as jnp
from jax import lax
from jax.experimental import pallas as pl
from jax.experimental.pallas import tpu as pltpu
from jax.experimental.pallas import tpu_sc as plsc

_N = 50000
_EH = 800000
_D = 16
_NC = 2
_NS = 16
_NW = _NC * _NS
_CHUNK = 128
_CPT = -(-_EH // (_NW * _CHUNK))
_EPT = _CPT * _CHUNK
_EP = _EPT * _NW
_RSEG = 3136
_NP = _RSEG * _NS
_SB = _RSEG // 8
_BL = 3136
_NBLK = _EPT // _BL
_SCN = 14
_NSUP = _CPT // _SCN
_SE = _SCN * _CHUNK
_F32 = jnp.float32
_I32 = jnp.int32


def _mesh():
    return plsc.VectorSubcoreMesh(
        core_axis_name="c", subcore_axis_name="s",
        num_cores=_NC, num_subcores=_NS)


def _sc_params():
    return pltpu.CompilerParams(
        needs_layout_passes=False, use_tc_tiling_on_sc=False)


def _rsqrt16(x):
    i = lax.bitcast_convert_type(x, _I32)
    i = jnp.int32(0x5F3759DF) - (i >> 1)
    r = lax.bitcast_convert_type(i, _F32)
    for _ in range(3):
        r = r * (1.5 - 0.5 * x * r * r)
    return r


def _deg_kernel(w1, w2, srcp, dstp):
    @functools.partial(
        pl.kernel,
        out_type=(jax.ShapeDtypeStruct((_EP,), _F32),
                  jax.ShapeDtypeStruct((_NC * _NP,), _F32)),
        mesh=_mesh(),
        scratch_types=[
            pltpu.VMEM((_BL,), _F32),
            pltpu.VMEM((_BL,), _F32),
            pltpu.VMEM((_BL,), _F32),
            pltpu.VMEM((_BL,), _I32),
            pltpu.VMEM((_BL,), _I32),
            pltpu.VMEM((_NP,), _F32),
            pltpu.VMEM((_RSEG,), _F32),
            pltpu.VMEM((_RSEG,), _F32),
            pltpu.VMEM_SHARED((_NS * _NP,), _F32),
            pltpu.SemaphoreType.DMA,
        ],
        compiler_params=_sc_params(),
    )
    def k(w1_ref, w2_ref, src_ref, dst_ref, wout_ref, degp_ref,
          w1b, w2b, wb, sb, db, degtile, tmpb, accb, degsh, semL):
        c = lax.axis_index("c")
        s = lax.axis_index("s")
        wid = c * _NS + s
        ebase = wid * _EPT
        zero16 = jnp.zeros((16,), _F32)
        iota16 = lax.iota(_I32, 16)

        @pl.loop(0, _NP, step=16)
        def _(i):
            degtile[pl.ds(i, 16)] = zero16

        @pl.loop(0, _NBLK)
        def _(blk):
            off = pl.multiple_of(ebase + blk * _BL, 8)
            sl = pl.ds(off, _BL)
            d1 = pltpu.async_copy(w1_ref.at[sl], w1b, semL)
            d2 = pltpu.async_copy(w2_ref.at[sl], w2b, semL)
            d3 = pltpu.async_copy(src_ref.at[sl], sb, semL)
            d4 = pltpu.async_copy(dst_ref.at[sl], db, semL)
            d1.wait(); d2.wait(); d3.wait(); d4.wait()

            @pl.loop(0, _BL, step=16)
            def _(g):
                a = w1b[pl.ds(g, 16)]
                b = w2b[pl.ds(g, 16)]
                wv = 1.0 / (1.0 + jnp.exp(-0.5 * (a + b)))
                gidx = off + g + iota16
                wv = jnp.where(gidx < _EH, wv, 0.0)
                wb[pl.ds(g, 16)] = wv
                plsc.addupdate_scatter(degtile, [db[pl.ds(g, 16)]], wv)
                plsc.addupdate_scatter(degtile, [sb[pl.ds(g, 16)]], wv)

            pltpu.sync_copy(wb, wout_ref.at[sl])

        pltpu.sync_copy(degtile, degsh.at[pl.ds(pl.multiple_of(s * _NP, 8), _NP)])
        plsc.subcore_barrier()
        rbase = s * _RSEG

        @pl.loop(0, _RSEG, step=16)
        def _(i):
            accb[pl.ds(i, 16)] = zero16

        @pl.loop(0, _NS)
        def _(j):
            joff = pl.multiple_of(j * _NP + rbase, 8)
            pltpu.sync_copy(degsh.at[pl.ds(joff, _RSEG)], tmpb)

            @pl.loop(0, _RSEG, step=16)
            def _(i):
                accb[pl.ds(i, 16)] = accb[pl.ds(i, 16)] + tmpb[pl.ds(i, 16)]

        pltpu.sync_copy(
            accb, degp_ref.at[pl.ds(pl.multiple_of(c * _NP + rbase, 8), _RSEG)])

    return k(w1, w2, srcp, dstp)


def _edge_kernel(ytp, degp, src3, dst3, w3):
    @functools.partial(
        pl.kernel,
        out_type=(jax.ShapeDtypeStruct((_NC * _NP, _D), _F32),
                  jax.ShapeDtypeStruct((_NP,), _F32),
                  jax.ShapeDtypeStruct((_NP, _D), _F32)),
        mesh=_mesh(),
        scratch_types=[
            pltpu.VMEM((_RSEG,), _F32),
            pltpu.VMEM((_RSEG,), _F32),
            pltpu.VMEM((_RSEG,), _F32),
            pltpu.VMEM((_SB, _D), _F32),
            pltpu.VMEM((_SCN, _CHUNK), _I32),
            pltpu.VMEM((_SCN, _CHUNK), _I32),
            pltpu.VMEM((_SCN, _CHUNK), _F32),
            pltpu.VMEM((_SE, _D), _F32),
            pltpu.VMEM((_SE, _D), _F32),
            pltpu.VMEM_SHARED((_NP, _D), _F32),
            pltpu.SemaphoreType.DMA,
            pltpu.SemaphoreType.DMA,
        ],
        compiler_params=_sc_params(),
    )
    def k(yt_ref, degp_ref, src_ref, dst_ref, w_ref, accp_ref, degc_ref,
          z_ref, p0b, p1b, rsb, yb, sb2, db2, wb2, rows_s, rows_d,
          accsh, semG, semS):
        c = lax.axis_index("c")
        s = lax.axis_index("s")
        rbase = s * _RSEG
        zero16 = jnp.zeros((16,), _F32)

        rb8 = pl.multiple_of(rbase, 8)
        pltpu.sync_copy(degp_ref.at[pl.ds(rb8, _RSEG)], p0b)
        pltpu.sync_copy(
            degp_ref.at[pl.ds(pl.multiple_of(_NP + rbase, 8), _RSEG)], p1b)

        @pl.loop(0, _RSEG, step=16)
        def _(i):
            dv = 1.0 + p0b[pl.ds(i, 16)] + p1b[pl.ds(i, 16)]
            p0b[pl.ds(i, 16)] = dv
            rsb[pl.ds(i, 16)] = _rsqrt16(dv)

        @pl.when(c == 0)
        def _():
            pltpu.sync_copy(p0b, degc_ref.at[pl.ds(rb8, _RSEG)])

        @pl.loop(0, _SB)
        def _(r):
            yb[r] = zero16

        for blk in range(_RSEG // _SB):
            bb = pl.multiple_of(rbase + blk * _SB, 8)
            pltpu.sync_copy(yb, accsh.at[pl.ds(bb, _SB)])

        for blk in range(_RSEG // _SB):
            bb = pl.multiple_of(rbase + blk * _SB, 8)
            pltpu.sync_copy(yt_ref.at[pl.ds(bb, _SB)], yb)

            @pl.loop(0, _SB, step=16)
            def _(g):
                rs16 = rsb[pl.ds(blk * _SB + g, 16)]
                for j in range(16):
                    yb[g + j] = yb[g + j] * rs16[j]

            pltpu.sync_copy(yb, z_ref.at[pl.ds(bb, _SB)])

        plsc.subcore_barrier()

        wid = c * _NS + s

        @pl.loop(0, _NSUP)
        def _(sup):
            csl = pl.ds(sup * _SCN, _SCN)
            pltpu.sync_copy(src_ref.at[wid, csl], sb2)
            pltpu.sync_copy(dst_ref.at[wid, csl], db2)
            pltpu.sync_copy(w_ref.at[wid, csl], wb2)

            gds = []
            for kk in range(_SCN):
                rsl = pl.ds(kk * _CHUNK, _CHUNK)
                gds.append(pltpu.async_copy(
                    z_ref.at[sb2.at[kk]], rows_s.at[rsl], semG))
                gds.append(pltpu.async_copy(
                    z_ref.at[db2.at[kk]], rows_d.at[rsl], semG))
            for d in gds:
                d.wait()

            @pl.loop(0, _SCN)
            def _(kk):
                @pl.loop(0, _CHUNK, step=16)
                def _(g):
                    wv16 = wb2[kk, pl.ds(g, 16)]
                    rr = kk * _CHUNK + g
                    for j in range(16):
                        wv = wv16[j]
                        rows_s[rr + j] = rows_s[rr + j] * wv
                        rows_d[rr + j] = rows_d[rr + j] * wv

            sds = []
            for kk in range(_SCN):
                rsl = pl.ds(kk * _CHUNK, _CHUNK)
                sds.append(pltpu.async_copy(
                    rows_s.at[rsl], accsh.at[db2.at[kk]], semS, add=True))
                sds.append(pltpu.async_copy(
                    rows_d.at[rsl], accsh.at[sb2.at[kk]], semS, add=True))
            for d in sds:
                d.wait()

        plsc.subcore_barrier()

        for blk in range(_RSEG // _SB):
            bb = pl.multiple_of(rbase + blk * _SB, 8)
            pltpu.sync_copy(accsh.at[pl.ds(bb, _SB)], yb)
            obb = pl.multiple_of(c * _NP + rbase + blk * _SB, 8)
            pltpu.sync_copy(yb, accp_ref.at[pl.ds(obb, _SB)])

    return k(ytp, degp, src3, dst3, w3)


def _combine(accp, degc, ytp):
    br = _NP // 8

    def body(acc_ref, deg_ref, y_ref, o_ref):
        a = acc_ref[0] + acc_ref[1]
        deg = deg_ref[...]
        rs = lax.rsqrt(deg)
        o_ref[...] = a * rs - (1.0 - 1.0 / deg) * y_ref[...]

    return pl.pallas_call(
        body,
        grid=(8,),
        in_specs=[
            pl.BlockSpec((_NC, br, _D), lambda i: (0, i, 0)),
            pl.BlockSpec((br, 1), lambda i: (i, 0)),
            pl.BlockSpec((br, _D), lambda i: (i, 0)),
        ],
        out_specs=pl.BlockSpec((br, _D), lambda i: (i, 0)),
        out_shape=jax.ShapeDtypeStruct((_NP, _D), _F32),
    )(accp.reshape(_NC, _NP, _D), degc.reshape(_NP, 1), ytp)


def kernel(t, y, W, edge_index, perm):
    del t, perm
    wf = W.reshape(-1)
    pad_e = _EP - _EH
    w1 = jnp.pad(wf[:_EH], (0, pad_e))
    w2 = jnp.pad(wf[_EH:], (0, pad_e))
    srcp = jnp.pad(edge_index[0, :_EH], (0, pad_e))
    dstp = jnp.pad(edge_index[1, :_EH], (0, pad_e))
    ytp = jnp.pad(y.T, ((0, _NP - _N), (0, 0)))
    w, degp = _deg_kernel(w1, w2, srcp, dstp)
    src3 = srcp.reshape(_NW, _CPT, _CHUNK)
    dst3 = dstp.reshape(_NW, _CPT, _CHUNK)
    w3 = w.reshape(_NW, _CPT, _CHUNK)
    accp, degc, _ = _edge_kernel(ytp, degp, src3, dst3, w3)
    res = _combine(accp, degc, ytp)
    return res[:_N].T


# speedup vs baseline: 63.8726x; 1.3246x over previous
import functools

import jax
import jax.numpy as jnp
from jax import lax
from jax.experimental import pallas as pl
from jax.experimental.pallas import tpu as pltpu
from jax.experimental.pallas import tpu_sc as plsc

_N = 50000
_EH = 800000
_D = 16
_NC = 2
_NS = 16
_NW = _NC * _NS
_CHUNK = 128
_CPT = -(-_EH // (_NW * _CHUNK))
_EPT = _CPT * _CHUNK
_EP = _EPT * _NW
_RSEG = 3136
_NP = _RSEG * _NS
_SB = _RSEG // 8
_BL = 3136
_NBLK = _EPT // _BL
_SCN = 14
_NSUP = _CPT // _SCN
_SE = _SCN * _CHUNK
_F32 = jnp.float32
_I32 = jnp.int32


def _mesh():
    return plsc.VectorSubcoreMesh(
        core_axis_name="c", subcore_axis_name="s",
        num_cores=_NC, num_subcores=_NS)


def _sc_params():
    return pltpu.CompilerParams(
        needs_layout_passes=False, use_tc_tiling_on_sc=False)


def _rsqrt16(x):
    i = lax.bitcast_convert_type(x, _I32)
    i = jnp.int32(0x5F3759DF) - (i >> 1)
    r = lax.bitcast_convert_type(i, _F32)
    for _ in range(3):
        r = r * (1.5 - 0.5 * x * r * r)
    return r


def _deg_kernel(w1, w2, srcp, dstp):
    @functools.partial(
        pl.kernel,
        out_type=(jax.ShapeDtypeStruct((_EP,), _F32),
                  jax.ShapeDtypeStruct((_NC * _NP,), _F32)),
        mesh=_mesh(),
        scratch_types=[
            pltpu.VMEM((_BL,), _F32),
            pltpu.VMEM((_BL,), _F32),
            pltpu.VMEM((_BL,), _F32),
            pltpu.VMEM((_BL,), _I32),
            pltpu.VMEM((_BL,), _I32),
            pltpu.VMEM((_NP,), _F32),
            pltpu.VMEM((_RSEG,), _F32),
            pltpu.VMEM((_RSEG,), _F32),
            pltpu.VMEM_SHARED((_NS * _NP,), _F32),
            pltpu.SemaphoreType.DMA,
        ],
        compiler_params=_sc_params(),
    )
    def k(w1_ref, w2_ref, src_ref, dst_ref, wout_ref, degp_ref,
          w1b, w2b, wb, sb, db, degtile, tmpb, accb, degsh, semL):
        c = lax.axis_index("c")
        s = lax.axis_index("s")
        wid = c * _NS + s
        ebase = wid * _EPT
        zero16 = jnp.zeros((16,), _F32)
        iota16 = lax.iota(_I32, 16)

        @pl.loop(0, _NP, step=16)
        def _(i):
            degtile[pl.ds(i, 16)] = zero16

        @pl.loop(0, _NBLK)
        def _(blk):
            off = pl.multiple_of(ebase + blk * _BL, 8)
            sl = pl.ds(off, _BL)
            d1 = pltpu.async_copy(w1_ref.at[sl], w1b, semL)
            d2 = pltpu.async_copy(w2_ref.at[sl], w2b, semL)
            d3 = pltpu.async_copy(src_ref.at[sl], sb, semL)
            d4 = pltpu.async_copy(dst_ref.at[sl], db, semL)
            d1.wait(); d2.wait(); d3.wait(); d4.wait()

            @pl.loop(0, _BL, step=16)
            def _(g):
                a = w1b[pl.ds(g, 16)]
                b = w2b[pl.ds(g, 16)]
                wv = 1.0 / (1.0 + jnp.exp(-0.5 * (a + b)))
                gidx = off + g + iota16
                wv = jnp.where(gidx < _EH, wv, 0.0)
                wb[pl.ds(g, 16)] = wv
                plsc.addupdate_scatter(degtile, [db[pl.ds(g, 16)]], wv)
                plsc.addupdate_scatter(degtile, [sb[pl.ds(g, 16)]], wv)

            pltpu.sync_copy(wb, wout_ref.at[sl])

        pltpu.sync_copy(degtile, degsh.at[pl.ds(pl.multiple_of(s * _NP, 8), _NP)])
        plsc.subcore_barrier()
        rbase = s * _RSEG

        @pl.loop(0, _RSEG, step=16)
        def _(i):
            accb[pl.ds(i, 16)] = zero16

        @pl.loop(0, _NS)
        def _(j):
            joff = pl.multiple_of(j * _NP + rbase, 8)
            pltpu.sync_copy(degsh.at[pl.ds(joff, _RSEG)], tmpb)

            @pl.loop(0, _RSEG, step=16)
            def _(i):
                accb[pl.ds(i, 16)] = accb[pl.ds(i, 16)] + tmpb[pl.ds(i, 16)]

        pltpu.sync_copy(
            accb, degp_ref.at[pl.ds(pl.multiple_of(c * _NP + rbase, 8), _RSEG)])

    return k(w1, w2, srcp, dstp)


def _edge_kernel(ytp, degp, src3, dst3, w3):
    @functools.partial(
        pl.kernel,
        out_type=(jax.ShapeDtypeStruct((_NC * _NP, _D), _F32),
                  jax.ShapeDtypeStruct((_NP,), _F32),
                  jax.ShapeDtypeStruct((_NP, _D), _F32)),
        mesh=_mesh(),
        scratch_types=[
            pltpu.VMEM((_RSEG,), _F32),
            pltpu.VMEM((_RSEG,), _F32),
            pltpu.VMEM((_RSEG,), _F32),
            pltpu.VMEM((_SB, _D), _F32),
            pltpu.VMEM((_SE,), _I32),
            pltpu.VMEM((_SE,), _I32),
            pltpu.VMEM((_SE,), _F32),
            pltpu.VMEM((_SE, _D), _F32),
            pltpu.VMEM((_SE, _D), _F32),
            pltpu.VMEM_SHARED((_NP, _D), _F32),
            pltpu.SemaphoreType.DMA,
            pltpu.SemaphoreType.DMA,
        ],
        compiler_params=_sc_params(),
    )
    def k(yt_ref, degp_ref, src_ref, dst_ref, w_ref, accp_ref, degc_ref,
          z_ref, p0b, p1b, rsb, yb, sb1, db1, wb1, rows_s, rows_d,
          accsh, semG, semS):
        c = lax.axis_index("c")
        s = lax.axis_index("s")
        rbase = s * _RSEG
        zero16 = jnp.zeros((16,), _F32)

        rb8 = pl.multiple_of(rbase, 8)
        pltpu.sync_copy(degp_ref.at[pl.ds(rb8, _RSEG)], p0b)
        pltpu.sync_copy(
            degp_ref.at[pl.ds(pl.multiple_of(_NP + rbase, 8), _RSEG)], p1b)

        @pl.loop(0, _RSEG, step=16)
        def _(i):
            dv = 1.0 + p0b[pl.ds(i, 16)] + p1b[pl.ds(i, 16)]
            p0b[pl.ds(i, 16)] = dv
            rsb[pl.ds(i, 16)] = _rsqrt16(dv)

        @pl.when(c == 0)
        def _():
            pltpu.sync_copy(p0b, degc_ref.at[pl.ds(rb8, _RSEG)])

        @pl.loop(0, _SB)
        def _(r):
            yb[r] = zero16

        for blk in range(_RSEG // _SB):
            bb = pl.multiple_of(rbase + blk * _SB, 8)
            pltpu.sync_copy(yb, accsh.at[pl.ds(bb, _SB)])

        for blk in range(_RSEG // _SB):
            bb = pl.multiple_of(rbase + blk * _SB, 8)
            pltpu.sync_copy(yt_ref.at[pl.ds(bb, _SB)], yb)

            @pl.loop(0, _SB, step=16)
            def _(g):
                rs16 = rsb[pl.ds(blk * _SB + g, 16)]
                for j in range(16):
                    yb[g + j] = yb[g + j] * rs16[j]

            pltpu.sync_copy(yb, z_ref.at[pl.ds(bb, _SB)])

        plsc.subcore_barrier()

        wid = c * _NS + s

        @pl.loop(0, _NSUP)
        def _(sup):
            esl = pl.ds(pl.multiple_of(sup * _SE, 8), _SE)
            pltpu.sync_copy(src_ref.at[wid, esl], sb1)
            pltpu.sync_copy(dst_ref.at[wid, esl], db1)
            pltpu.sync_copy(w_ref.at[wid, esl], wb1)

            g1 = pltpu.async_copy(z_ref.at[sb1], rows_s, semG)
            g2 = pltpu.async_copy(z_ref.at[db1], rows_d, semG)
            g1.wait()
            g2.wait()

            @pl.loop(0, _SE, step=16)
            def _(g):
                wv16 = wb1[pl.ds(g, 16)]
                for j in range(16):
                    wv = wv16[j]
                    rows_s[g + j] = rows_s[g + j] * wv
                    rows_d[g + j] = rows_d[g + j] * wv

            s1 = pltpu.async_copy(rows_s, accsh.at[db1], semS, add=True)
            s2 = pltpu.async_copy(rows_d, accsh.at[sb1], semS, add=True)
            s1.wait()
            s2.wait()

        plsc.subcore_barrier()

        for blk in range(_RSEG // _SB):
            bb = pl.multiple_of(rbase + blk * _SB, 8)
            pltpu.sync_copy(accsh.at[pl.ds(bb, _SB)], yb)
            obb = pl.multiple_of(c * _NP + rbase + blk * _SB, 8)
            pltpu.sync_copy(yb, accp_ref.at[pl.ds(obb, _SB)])

    return k(ytp, degp, src3, dst3, w3)


def _combine(accp, degc, ytp):
    br = _NP // 8

    def body(acc_ref, deg_ref, y_ref, o_ref):
        a = acc_ref[0] + acc_ref[1]
        deg = deg_ref[...]
        rs = lax.rsqrt(deg)
        o_ref[...] = a * rs - (1.0 - 1.0 / deg) * y_ref[...]

    return pl.pallas_call(
        body,
        grid=(8,),
        in_specs=[
            pl.BlockSpec((_NC, br, _D), lambda i: (0, i, 0)),
            pl.BlockSpec((br, 1), lambda i: (i, 0)),
            pl.BlockSpec((br, _D), lambda i: (i, 0)),
        ],
        out_specs=pl.BlockSpec((br, _D), lambda i: (i, 0)),
        out_shape=jax.ShapeDtypeStruct((_NP, _D), _F32),
    )(accp.reshape(_NC, _NP, _D), degc.reshape(_NP, 1), ytp)


def kernel(t, y, W, edge_index, perm):
    del t, perm
    wf = W.reshape(-1)
    pad_e = _EP - _EH
    w1 = jnp.pad(wf[:_EH], (0, pad_e))
    w2 = jnp.pad(wf[_EH:], (0, pad_e))
    srcp = jnp.pad(edge_index[0, :_EH], (0, pad_e))
    dstp = jnp.pad(edge_index[1, :_EH], (0, pad_e))
    ytp = jnp.pad(y.T, ((0, _NP - _N), (0, 0)))
    w, degp = _deg_kernel(w1, w2, srcp, dstp)
    src3 = srcp.reshape(_NW, _EPT)
    dst3 = dstp.reshape(_NW, _EPT)
    w3 = w.reshape(_NW, _EPT)
    accp, degc, _ = _edge_kernel(ytp, degp, src3, dst3, w3)
    res = _combine(accp, degc, ytp)
    return res[:_N].T


# speedup vs baseline: 83.2824x; 1.3039x over previous
import functools

import jax
import jax.numpy as jnp
from jax import lax
from jax.experimental import pallas as pl
from jax.experimental.pallas import tpu as pltpu
from jax.experimental.pallas import tpu_sc as plsc

_N = 50000
_EH = 800000
_D = 16
_NC = 2
_NS = 16
_NW = _NC * _NS
_EPT = 25600
_EP = _EPT * _NW
_RSEG = 3136
_NP = _RSEG * _NS
_SB = _RSEG // 8
_BL = 1600
_NBLK = _EPT // _BL
_SE = 1600
_NSUP = _EPT // _SE
_F32 = jnp.float32
_I32 = jnp.int32


def _mesh():
    return plsc.VectorSubcoreMesh(
        core_axis_name="c", subcore_axis_name="s",
        num_cores=_NC, num_subcores=_NS)


def _sc_params():
    return pltpu.CompilerParams(
        needs_layout_passes=False, use_tc_tiling_on_sc=False)


def _rsqrt16(x):
    i = lax.bitcast_convert_type(x, _I32)
    i = jnp.int32(0x5F3759DF) - (i >> 1)
    r = lax.bitcast_convert_type(i, _F32)
    for _ in range(3):
        r = r * (1.5 - 0.5 * x * r * r)
    return r


def _deg_kernel(W, edge_index):
    @functools.partial(
        pl.kernel,
        out_type=(jax.ShapeDtypeStruct((_EP,), _F32),
                  jax.ShapeDtypeStruct((_NC * _NP,), _F32)),
        mesh=_mesh(),
        scratch_types=[
            pltpu.VMEM((1, _BL), _F32),
            pltpu.VMEM((1, _BL), _F32),
            pltpu.VMEM((_BL,), _F32),
            pltpu.VMEM((1, _BL), _I32),
            pltpu.VMEM((1, _BL), _I32),
            pltpu.VMEM((_NP,), _F32),
            pltpu.VMEM((_RSEG,), _F32),
            pltpu.VMEM((_RSEG,), _F32),
            pltpu.VMEM_SHARED((_NS * _NP,), _F32),
            pltpu.SemaphoreType.DMA,
        ],
        compiler_params=_sc_params(),
    )
    def k(w_ref, ei_ref, wout_ref, degp_ref,
          w1b, w2b, wb, sb, db, degtile, tmpb, accb, degsh, semL):
        c = lax.axis_index("c")
        s = lax.axis_index("s")
        wid = c * _NS + s
        ebase = wid * _EPT
        zero16 = jnp.zeros((16,), _F32)

        @pl.loop(0, _NP, step=16)
        def _(i):
            degtile[pl.ds(i, 16)] = zero16

        @pl.loop(0, _NBLK)
        def _(blk):
            off = pl.multiple_of(ebase + blk * _BL, 8)
            sl = pl.ds(off, _BL)

            @pl.when(off < _EH)
            def _():
                d1 = pltpu.async_copy(w_ref.at[pl.ds(0, 1), sl], w1b, semL)
                d2 = pltpu.async_copy(w_ref.at[pl.ds(1, 1), sl], w2b, semL)
                d3 = pltpu.async_copy(ei_ref.at[pl.ds(0, 1), sl], sb, semL)
                d4 = pltpu.async_copy(ei_ref.at[pl.ds(1, 1), sl], db, semL)
                d1.wait(); d2.wait(); d3.wait(); d4.wait()

                @pl.loop(0, _BL, step=16)
                def _(g):
                    a = w1b[0, pl.ds(g, 16)]
                    b = w2b[0, pl.ds(g, 16)]
                    wv = 1.0 / (1.0 + jnp.exp(-0.5 * (a + b)))
                    wb[pl.ds(g, 16)] = wv
                    plsc.addupdate_scatter(degtile, [db[0, pl.ds(g, 16)]], wv)
                    plsc.addupdate_scatter(degtile, [sb[0, pl.ds(g, 16)]], wv)

            @pl.when(off >= _EH)
            def _():
                @pl.loop(0, _BL, step=16)
                def _(g):
                    wb[pl.ds(g, 16)] = zero16

            pltpu.sync_copy(wb, wout_ref.at[sl])

        pltpu.sync_copy(degtile, degsh.at[pl.ds(pl.multiple_of(s * _NP, 8), _NP)])
        plsc.subcore_barrier()
        rbase = s * _RSEG

        @pl.loop(0, _RSEG, step=16)
        def _(i):
            accb[pl.ds(i, 16)] = zero16

        @pl.loop(0, _NS)
        def _(j):
            joff = pl.multiple_of(j * _NP + rbase, 8)
            pltpu.sync_copy(degsh.at[pl.ds(joff, _RSEG)], tmpb)

            @pl.loop(0, _RSEG, step=16)
            def _(i):
                accb[pl.ds(i, 16)] = accb[pl.ds(i, 16)] + tmpb[pl.ds(i, 16)]

        pltpu.sync_copy(
            accb, degp_ref.at[pl.ds(pl.multiple_of(c * _NP + rbase, 8), _RSEG)])

    return k(W.reshape(2, _EH), edge_index)


def _edge_kernel(ytp, degp, edge_index, w3):
    @functools.partial(
        pl.kernel,
        out_type=(jax.ShapeDtypeStruct((_NC * _NP, _D), _F32),
                  jax.ShapeDtypeStruct((_NP, _D), _F32),
                  jax.ShapeDtypeStruct((_NP, _D), _F32)),
        mesh=_mesh(),
        scratch_types=[
            pltpu.VMEM((_RSEG,), _F32),
            pltpu.VMEM((_RSEG,), _F32),
            pltpu.VMEM((_RSEG,), _F32),
            pltpu.VMEM((_SB, _D), _F32),
            pltpu.VMEM((1, _SE), _I32),
            pltpu.VMEM((1, _SE), _I32),
            pltpu.VMEM((_SE,), _F32),
            pltpu.VMEM((_SE, _D), _F32),
            pltpu.VMEM((_SE, _D), _F32),
            pltpu.VMEM_SHARED((_NP, _D), _F32),
            pltpu.SemaphoreType.DMA,
            pltpu.SemaphoreType.DMA,
        ],
        compiler_params=_sc_params(),
    )
    def k(yt_ref, degp_ref, ei_ref, w_ref, accp_ref, degx_ref,
          z_ref, p0b, p1b, rsb, yb, sb1, db1, wb1, rows_s, rows_d,
          accsh, semG, semS):
        c = lax.axis_index("c")
        s = lax.axis_index("s")
        rbase = s * _RSEG
        zero16 = jnp.zeros((16,), _F32)

        rb8 = pl.multiple_of(rbase, 8)
        pltpu.sync_copy(degp_ref.at[pl.ds(rb8, _RSEG)], p0b)
        pltpu.sync_copy(
            degp_ref.at[pl.ds(pl.multiple_of(_NP + rbase, 8), _RSEG)], p1b)

        @pl.loop(0, _RSEG, step=16)
        def _(i):
            dv = 1.0 + p0b[pl.ds(i, 16)] + p1b[pl.ds(i, 16)]
            p0b[pl.ds(i, 16)] = dv
            rsb[pl.ds(i, 16)] = _rsqrt16(dv)

        @pl.when(c == 0)
        def _():
            for blk in range(_RSEG // _SB):
                bb = pl.multiple_of(rbase + blk * _SB, 8)

                @pl.loop(0, _SB, step=16)
                def _(g):
                    dv = p0b[pl.ds(blk * _SB + g, 16)]
                    for j in range(16):
                        yb[g + j] = jnp.full((16,), dv[j], _F32)

                pltpu.sync_copy(yb, degx_ref.at[pl.ds(bb, _SB)])

        @pl.loop(0, _SB)
        def _(r):
            yb[r] = zero16

        for blk in range(_RSEG // _SB):
            bb = pl.multiple_of(rbase + blk * _SB, 8)
            pltpu.sync_copy(yb, accsh.at[pl.ds(bb, _SB)])

        for blk in range(_RSEG // _SB):
            bb = pl.multiple_of(rbase + blk * _SB, 8)
            pltpu.sync_copy(yt_ref.at[pl.ds(bb, _SB)], yb)

            @pl.loop(0, _SB, step=16)
            def _(g):
                rs16 = rsb[pl.ds(blk * _SB + g, 16)]
                for j in range(16):
                    yb[g + j] = yb[g + j] * rs16[j]

            pltpu.sync_copy(yb, z_ref.at[pl.ds(bb, _SB)])

        plsc.subcore_barrier()

        wid = c * _NS + s

        ebase = wid * _EPT

        @pl.loop(0, _NSUP)
        def _(sup):
            eoff = pl.multiple_of(ebase + sup * _SE, 8)
            pltpu.sync_copy(ei_ref.at[pl.ds(0, 1), pl.ds(eoff, _SE)], sb1)
            pltpu.sync_copy(ei_ref.at[pl.ds(1, 1), pl.ds(eoff, _SE)], db1)
            pltpu.sync_copy(
                w_ref.at[wid, pl.ds(pl.multiple_of(sup * _SE, 8), _SE)], wb1)

            g1 = pltpu.async_copy(z_ref.at[sb1.at[0]], rows_s, semG)
            g2 = pltpu.async_copy(z_ref.at[db1.at[0]], rows_d, semG)
            g1.wait()
            g2.wait()

            @pl.loop(0, _SE, step=16)
            def _(g):
                wv16 = wb1[pl.ds(g, 16)]
                for j in range(16):
                    wv = wv16[j]
                    rows_s[g + j] = rows_s[g + j] * wv
                    rows_d[g + j] = rows_d[g + j] * wv

            s1 = pltpu.async_copy(rows_s, accsh.at[db1.at[0]], semS, add=True)
            s2 = pltpu.async_copy(rows_d, accsh.at[sb1.at[0]], semS, add=True)
            s1.wait()
            s2.wait()

        plsc.subcore_barrier()

        for blk in range(_RSEG // _SB):
            bb = pl.multiple_of(rbase + blk * _SB, 8)
            pltpu.sync_copy(accsh.at[pl.ds(bb, _SB)], yb)
            obb = pl.multiple_of(c * _NP + rbase + blk * _SB, 8)
            pltpu.sync_copy(yb, accp_ref.at[pl.ds(obb, _SB)])

    return k(ytp, degp, edge_index, w3)


def _combine(accp, degx, ytp):
    nr = _NP * _D // 128
    br = nr // 8

    def body(acc_ref, deg_ref, y_ref, o_ref):
        a = acc_ref[0] + acc_ref[1]
        deg = deg_ref[...]
        rs = lax.rsqrt(deg)
        o_ref[...] = a * rs - (1.0 - 1.0 / deg) * y_ref[...]

    return pl.pallas_call(
        body,
        grid=(8,),
        in_specs=[
            pl.BlockSpec((_NC, br, 128), lambda i: (0, i, 0)),
            pl.BlockSpec((br, 128), lambda i: (i, 0)),
            pl.BlockSpec((br, 128), lambda i: (i, 0)),
        ],
        out_specs=pl.BlockSpec((br, 128), lambda i: (i, 0)),
        out_shape=jax.ShapeDtypeStruct((nr, 128), _F32),
    )(accp.reshape(_NC, nr, 128), degx.reshape(nr, 128), ytp.reshape(nr, 128))


def kernel(t, y, W, edge_index, perm):
    del t, perm
    ytp = jnp.pad(y.T, ((0, _NP - _N), (0, 0)))
    w, degp = _deg_kernel(W, edge_index)
    w3 = w.reshape(_NW, _EPT)
    accp, degx, _ = _edge_kernel(ytp, degp, edge_index, w3)
    res = _combine(accp, degx, ytp).reshape(_NP, _D)
    return res[:_N].T
